# baseline (speedup 1.0000x reference)
import functools

import jax
import jax.numpy as jnp
from jax import lax
from jax.experimental import pallas as pl
from jax.experimental.pallas import tpu as pltpu
from jax.experimental.pallas import tpu_sc as plsc

BATCH = 16384
EMBED = 16
ROWS_PER_LINE = 8
NW = 32
BPW = BATCH // NW
CHUNK = 128
NCH = BPW // CHUNK
NGRP = CHUNK // 16


def _extract_chunk(idx_v, c, rows_buf, out_v):
    lanes = lax.iota(jnp.int32, 16)
    for g in range(NGRP):
        raw = idx_v[pl.ds(c * CHUNK + g * 16, 16)]
        col0 = (raw & (ROWS_PER_LINE - 1)) * EMBED
        rows = g * 16 + lanes
        base_flat = (c * CHUNK + rows) * EMBED
        for k in range(EMBED):
            v = plsc.load_gather(rows_buf, [rows, col0 + k])
            plsc.store_scatter(out_v, [base_flat + k], v)


def _gather_body(user_hbm, item_hbm, utab_hbm, itab_hbm, uout_hbm, iout_hbm,
                 uidx_v, iidx_v, uhi_v, ihi_v, urows_v, irows_v,
                 uout_v, iout_v, sem):
    wid = lax.axis_index("s") * 2 + lax.axis_index("c")
    base = wid * BPW
    pltpu.sync_copy(user_hbm.at[pl.ds(base, BPW)], uidx_v)
    pltpu.sync_copy(item_hbm.at[pl.ds(base, BPW)], iidx_v)
    shift = ROWS_PER_LINE.bit_length() - 1
    for c in range(NCH):
        for g in range(NGRP):
            s = pl.ds(c * CHUNK + g * 16, 16)
            d = pl.ds(g * 16, 16)
            uhi_v[c, d] = lax.shift_right_logical(uidx_v[s], shift)
            ihi_v[c, d] = lax.shift_right_logical(iidx_v[s], shift)

    def fire(c):
        buf = c % 2
        return (pltpu.async_copy(utab_hbm.at[uhi_v.at[c]], urows_v.at[buf], sem),
                pltpu.async_copy(itab_hbm.at[ihi_v.at[c]], irows_v.at[buf], sem))

    pending = fire(0)
    for c in range(NCH):
        nxt = fire(c + 1) if c + 1 < NCH else None
        for cp in pending:
            cp.wait()
        buf = c % 2
        _extract_chunk(uidx_v, c, urows_v.at[buf], uout_v)
        _extract_chunk(iidx_v, c, irows_v.at[buf], iout_v)
        pending = nxt
    pltpu.sync_copy(uout_v, uout_hbm.at[pl.ds(base * EMBED, BPW * EMBED)])
    pltpu.sync_copy(iout_v, iout_hbm.at[pl.ds(base * EMBED, BPW * EMBED)])


@functools.cache
def _gather():
    return pl.kernel(
        _gather_body,
        mesh=plsc.VectorSubcoreMesh(core_axis_name="c", subcore_axis_name="s"),
        compiler_params=pltpu.CompilerParams(needs_layout_passes=False),
        out_type=[
            jax.ShapeDtypeStruct((BATCH * EMBED,), jnp.float32),
            jax.ShapeDtypeStruct((BATCH * EMBED,), jnp.float32),
        ],
        scratch_types=[
            pltpu.VMEM((BPW,), jnp.int32),
            pltpu.VMEM((BPW,), jnp.int32),
            pltpu.VMEM((NCH, CHUNK), jnp.int32),
            pltpu.VMEM((NCH, CHUNK), jnp.int32),
            pltpu.VMEM((2, CHUNK, 128), jnp.float32),
            pltpu.VMEM((2, CHUNK, 128), jnp.float32),
            pltpu.VMEM((BPW * EMBED,), jnp.float32),
            pltpu.VMEM((BPW * EMBED,), jnp.float32),
            pltpu.SemaphoreType.DMA,
        ],
    )


B_BLK = 2048


def _mlp_body(u_ref, i_ref, w1u_ref, w1i_ref, b1_ref, w2_ref, b2_ref,
              w3_ref, b3_ref, out_ref):
    h = (jnp.dot(u_ref[...], w1u_ref[...], preferred_element_type=jnp.float32)
         + jnp.dot(i_ref[...], w1i_ref[...], preferred_element_type=jnp.float32)
         + b1_ref[...])
    h = jnp.maximum(h, 0.0)
    h = jnp.dot(h, w2_ref[...], preferred_element_type=jnp.float32) + b2_ref[...]
    h = jnp.maximum(h, 0.0)
    out_ref[...] = (jnp.dot(h, w3_ref[...], preferred_element_type=jnp.float32)
                    + b3_ref[...])


def _mlp(u_emb, i_emb, W1u, W1i, b1, W2, b2, W3, b3):
    grid = (BATCH // B_BLK,)
    full = lambda shape: pl.BlockSpec(shape, lambda i: (0, 0))
    return pl.pallas_call(
        _mlp_body,
        grid=grid,
        in_specs=[
            pl.BlockSpec((B_BLK, EMBED), lambda i: (i, 0)),
            pl.BlockSpec((B_BLK, EMBED), lambda i: (i, 0)),
            full((EMBED, 16)),
            full((EMBED, 16)),
            full((1, 16)),
            full((16, 8)),
            full((1, 8)),
            full((8, 1)),
            full((1, 1)),
        ],
        out_specs=pl.BlockSpec((B_BLK, 1), lambda i: (i, 0)),
        out_shape=jax.ShapeDtypeStruct((BATCH, 1), jnp.float32),
    )(u_emb, i_emb, W1u, W1i, b1, W2, b2, W3, b3)


def kernel(user, item, user_table, item_table, W1, b1, W2, b2, W3, b3):
    utab = user_table.reshape(-1, ROWS_PER_LINE * EMBED)
    itab = item_table.reshape(-1, ROWS_PER_LINE * EMBED)
    u_flat, i_flat = _gather()(user.astype(jnp.int32), item.astype(jnp.int32),
                               utab, itab)
    u_emb = u_flat.reshape(BATCH, EMBED)
    i_emb = i_flat.reshape(BATCH, EMBED)
    out = _mlp(u_emb, i_emb,
               W1[:EMBED], W1[EMBED:], b1.reshape(1, 16),
               W2, b2.reshape(1, 8), W3, b3.reshape(1, 1))
    return out[:, 0]


# speedup vs baseline: 4.2862x; 4.2862x over previous
import functools

import jax
import jax.numpy as jnp
from jax import lax
from jax.experimental import pallas as pl
from jax.experimental.pallas import tpu as pltpu
from jax.experimental.pallas import tpu_sc as plsc

BATCH = 16384
EMBED = 16
NW = 32
BPW = BATCH // NW
NGRP = BPW // 16
DEPTH = 8


def _gather_body(user_hbm, item_hbm, uttab_hbm, ittab_hbm, uout_hbm, iout_hbm,
                 uidx_v, iidx_v, slab_v, uflat_v, iflat_v, *sems):
    wid = lax.axis_index("s") * 2 + lax.axis_index("c")
    base = wid * BPW
    pltpu.sync_copy(user_hbm.at[pl.ds(base, BPW)], uidx_v)
    pltpu.sync_copy(item_hbm.at[pl.ds(base, BPW)], iidx_v)
    lanes = lax.iota(jnp.int32, 16)

    for t, (idx_v, tab_hbm, flat_v) in enumerate(
        ((uidx_v, uttab_hbm, uflat_v), (iidx_v, ittab_hbm, iflat_v))):
        sem = sems[t * DEPTH:(t + 1) * DEPTH]

        def grp(g, carry, idx_v=idx_v, tab_hbm=tab_hbm, flat_v=flat_v, sem=sem):
            iv = idx_v[pl.ds(g * 16, 16)]
            col = iv & 127
            off = (iv >> 7) * 128
            for half in range(2):
                for k in range(DEPTH):
                    e = half * DEPTH + k
                    pltpu.async_copy(
                        tab_hbm.at[:, pl.ds(pl.multiple_of(off[e], 128), 128)],
                        slab_v.at[t * 2 * DEPTH + half * DEPTH + k], sem[k])
                for k in range(DEPTH):
                    e = half * DEPTH + k
                    pltpu.make_async_copy(
                        tab_hbm.at[:, pl.ds(0, 128)],
                        slab_v.at[t * 2 * DEPTH + half * DEPTH + k],
                        sem[k]).wait()
                    v = plsc.load_gather(
                        slab_v.at[t * 2 * DEPTH + half * DEPTH + k],
                        [lanes, jnp.full((16,), col[e], jnp.int32)])
                    plsc.store_scatter(
                        flat_v, [(g * 16 + e) * EMBED + lanes], v)
            return carry

        lax.fori_loop(0, NGRP, grp, 0)

    pltpu.sync_copy(uflat_v, uout_hbm.at[pl.ds(base * EMBED, BPW * EMBED)])
    pltpu.sync_copy(iflat_v, iout_hbm.at[pl.ds(base * EMBED, BPW * EMBED)])


@functools.cache
def _gather():
    return pl.kernel(
        _gather_body,
        mesh=plsc.VectorSubcoreMesh(core_axis_name="c", subcore_axis_name="s"),
        compiler_params=pltpu.CompilerParams(needs_layout_passes=False),
        out_type=[
            jax.ShapeDtypeStruct((BATCH * EMBED,), jnp.float32),
            jax.ShapeDtypeStruct((BATCH * EMBED,), jnp.float32),
        ],
        scratch_types=(
            [pltpu.VMEM((BPW,), jnp.int32),
             pltpu.VMEM((BPW,), jnp.int32),
             pltpu.VMEM((4 * DEPTH, EMBED, 128), jnp.float32),
             pltpu.VMEM((BPW * EMBED,), jnp.float32),
             pltpu.VMEM((BPW * EMBED,), jnp.float32)]
            + [pltpu.SemaphoreType.DMA] * (2 * DEPTH)
        ),
    )


B_BLK = 2048
R_BLK = B_BLK // 8


def _mlp_body(xu_ref, xi_ref, m1u_ref, m1i_ref, b1_ref, m2_ref, b2_ref,
              m3_ref, b3_ref, out_ref):
    h = (jnp.dot(xu_ref[...], m1u_ref[...], preferred_element_type=jnp.float32)
         + jnp.dot(xi_ref[...], m1i_ref[...], preferred_element_type=jnp.float32)
         + b1_ref[...])
    h = jnp.maximum(h, 0.0)
    h = jnp.dot(h, m2_ref[...], preferred_element_type=jnp.float32) + b2_ref[...]
    h = jnp.maximum(h, 0.0)
    out_ref[...] = (jnp.dot(h, m3_ref[...], preferred_element_type=jnp.float32)
                    + b3_ref[...])


def _mlp(xu, xi, m1u, m1i, b1t, m2, b2t, m3, b3t):
    grid = (BATCH // B_BLK,)
    full = lambda shape: pl.BlockSpec(shape, lambda i: (0, 0))
    return pl.pallas_call(
        _mlp_body,
        grid=grid,
        in_specs=[
            pl.BlockSpec((R_BLK, 128), lambda i: (i, 0)),
            pl.BlockSpec((R_BLK, 128), lambda i: (i, 0)),
            full((128, 128)),
            full((128, 128)),
            full((1, 128)),
            full((128, 64)),
            full((1, 64)),
            full((64, 8)),
            full((1, 8)),
        ],
        out_specs=pl.BlockSpec((R_BLK, 8), lambda i: (i, 0)),
        out_shape=jax.ShapeDtypeStruct((BATCH // 8, 8), jnp.float32),
    )(xu, xi, m1u, m1i, b1t, m2, b2t, m3, b3t)


def kernel(user, item, user_table, item_table, W1, b1, W2, b2, W3, b3):
    u_flat, i_flat = _gather()(user.astype(jnp.int32), item.astype(jnp.int32),
                               user_table.T, item_table.T)
    xu = u_flat.reshape(BATCH // 8, 128)
    xi = i_flat.reshape(BATCH // 8, 128)
    eye8 = jnp.eye(8, dtype=jnp.float32)
    m1u = jnp.kron(eye8, W1[:EMBED])
    m1i = jnp.kron(eye8, W1[EMBED:])
    b1t = jnp.tile(b1, 8).reshape(1, 128)
    m2 = jnp.kron(eye8, W2)
    b2t = jnp.tile(b2, 8).reshape(1, 64)
    m3 = jnp.kron(eye8, W3)
    b3t = jnp.tile(b3, 8).reshape(1, 8)
    out = _mlp(xu, xi, m1u, m1i, b1t, m2, b2t, m3, b3t)
    return out.reshape(BATCH)


# speedup vs baseline: 4.5202x; 1.0546x over previous
import functools

import jax
import jax.numpy as jnp
from jax import lax
from jax.experimental import pallas as pl
from jax.experimental.pallas import tpu as pltpu
from jax.experimental.pallas import tpu_sc as plsc

BATCH = 16384
EMBED = 16
NW = 32
BPW = BATCH // NW
NGRP = BPW // 16
DEPTH = 8


def _gather_body(user_hbm, item_hbm, uttab_hbm, ittab_hbm, uout_hbm, iout_hbm,
                 uidx_v, iidx_v, slab_v, uflat_v, iflat_v, *sems):
    wid = lax.axis_index("s") * 2 + lax.axis_index("c")
    base = wid * BPW
    pltpu.sync_copy(user_hbm.at[pl.ds(base, BPW)], uidx_v)
    pltpu.sync_copy(item_hbm.at[pl.ds(base, BPW)], iidx_v)
    lanes = lax.iota(jnp.int32, 16)

    for t, (idx_v, tab_hbm, flat_v) in enumerate(
        ((uidx_v, uttab_hbm, uflat_v), (iidx_v, ittab_hbm, iflat_v))):

        def grp(g, carry, idx_v=idx_v, tab_hbm=tab_hbm, flat_v=flat_v):
            iv = idx_v[pl.ds(g * 16, 16)]
            col = iv & 127
            off = (iv >> 7) * 128
            for k in range(16):
                pltpu.async_copy(
                    tab_hbm.at[:, pl.ds(pl.multiple_of(off[k], 128), 128)],
                    slab_v.at[t * 16 + k], sems[k])
            for k in range(16):
                pltpu.make_async_copy(
                    tab_hbm.at[:, pl.ds(0, 128)],
                    slab_v.at[t * 16 + k], sems[k]).wait()
                v = plsc.load_gather(
                    slab_v.at[t * 16 + k],
                    [lanes, jnp.full((16,), col[k], jnp.int32)])
                plsc.store_scatter(flat_v, [(g * 16 + k) * EMBED + lanes], v)
            return carry

        lax.fori_loop(0, NGRP, grp, 0)

    pltpu.sync_copy(uflat_v, uout_hbm.at[pl.ds(base * EMBED, BPW * EMBED)])
    pltpu.sync_copy(iflat_v, iout_hbm.at[pl.ds(base * EMBED, BPW * EMBED)])


@functools.cache
def _gather():
    return pl.kernel(
        _gather_body,
        mesh=plsc.VectorSubcoreMesh(core_axis_name="c", subcore_axis_name="s"),
        compiler_params=pltpu.CompilerParams(needs_layout_passes=False),
        out_type=[
            jax.ShapeDtypeStruct((BATCH * EMBED,), jnp.float32),
            jax.ShapeDtypeStruct((BATCH * EMBED,), jnp.float32),
        ],
        scratch_types=(
            [pltpu.VMEM((BPW,), jnp.int32),
             pltpu.VMEM((BPW,), jnp.int32),
             pltpu.VMEM((4 * DEPTH, EMBED, 128), jnp.float32),
             pltpu.VMEM((BPW * EMBED,), jnp.float32),
             pltpu.VMEM((BPW * EMBED,), jnp.float32)]
            + [pltpu.SemaphoreType.DMA] * (2 * DEPTH)
        ),
    )


B_BLK = 2048
R_BLK = B_BLK // 8


def _mlp_body(xu_ref, xi_ref, m1u_ref, m1i_ref, b1_ref, m2_ref, b2_ref,
              m3_ref, b3_ref, out_ref):
    h = (jnp.dot(xu_ref[...], m1u_ref[...], preferred_element_type=jnp.float32)
         + jnp.dot(xi_ref[...], m1i_ref[...], preferred_element_type=jnp.float32)
         + b1_ref[...])
    h = jnp.maximum(h, 0.0)
    h = jnp.dot(h, m2_ref[...], preferred_element_type=jnp.float32) + b2_ref[...]
    h = jnp.maximum(h, 0.0)
    out_ref[...] = (jnp.dot(h, m3_ref[...], preferred_element_type=jnp.float32)
                    + b3_ref[...])


def _mlp(xu, xi, m1u, m1i, b1t, m2, b2t, m3, b3t):
    grid = (BATCH // B_BLK,)
    full = lambda shape: pl.BlockSpec(shape, lambda i: (0, 0))
    return pl.pallas_call(
        _mlp_body,
        grid=grid,
        in_specs=[
            pl.BlockSpec((R_BLK, 128), lambda i: (i, 0)),
            pl.BlockSpec((R_BLK, 128), lambda i: (i, 0)),
            full((128, 128)),
            full((128, 128)),
            full((1, 128)),
            full((128, 64)),
            full((1, 64)),
            full((64, 8)),
            full((1, 8)),
        ],
        out_specs=pl.BlockSpec((R_BLK, 8), lambda i: (i, 0)),
        out_shape=jax.ShapeDtypeStruct((BATCH // 8, 8), jnp.float32),
    )(xu, xi, m1u, m1i, b1t, m2, b2t, m3, b3t)


def kernel(user, item, user_table, item_table, W1, b1, W2, b2, W3, b3):
    u_flat, i_flat = _gather()(user.astype(jnp.int32), item.astype(jnp.int32),
                               user_table.T, item_table.T)
    xu = u_flat.reshape(BATCH // 8, 128)
    xi = i_flat.reshape(BATCH // 8, 128)
    eye8 = jnp.eye(8, dtype=jnp.float32)
    m1u = jnp.kron(eye8, W1[:EMBED])
    m1i = jnp.kron(eye8, W1[EMBED:])
    b1t = jnp.tile(b1, 8).reshape(1, 128)
    m2 = jnp.kron(eye8, W2)
    b2t = jnp.tile(b2, 8).reshape(1, 64)
    m3 = jnp.kron(eye8, W3)
    b3t = jnp.tile(b3, 8).reshape(1, 8)
    out = _mlp(xu, xi, m1u, m1i, b1t, m2, b2t, m3, b3t)
    return out.reshape(BATCH)


# speedup vs baseline: 4.6071x; 1.0192x over previous
import functools

import jax
import jax.numpy as jnp
from jax import lax
from jax.experimental import pallas as pl
from jax.experimental.pallas import tpu as pltpu
from jax.experimental.pallas import tpu_sc as plsc

BATCH = 16384
EMBED = 16
NW = 32
BPW = BATCH // NW
NGRP = BPW // 16
DEPTH = 8


def _gather_body(user_hbm, item_hbm, uttab_hbm, ittab_hbm, uout_hbm, iout_hbm,
                 uidx_v, iidx_v, slab_v, uflat_v, iflat_v, *sems):
    wid = lax.axis_index("s") * 2 + lax.axis_index("c")
    base = wid * BPW
    pltpu.sync_copy(user_hbm.at[pl.ds(base, BPW)], uidx_v)
    pltpu.sync_copy(item_hbm.at[pl.ds(base, BPW)], iidx_v)
    lanes = lax.iota(jnp.int32, 16)

    for t, (idx_v, tab_hbm, flat_v) in enumerate(
        ((uidx_v, uttab_hbm, uflat_v), (iidx_v, ittab_hbm, iflat_v))):

        def grp(g, carry, idx_v=idx_v, tab_hbm=tab_hbm, flat_v=flat_v):
            iv = idx_v[pl.ds(g * 16, 16)]
            col = iv & 127
            off = (iv >> 7) * 128
            for k in range(16):
                pltpu.async_copy(
                    tab_hbm.at[:, pl.ds(pl.multiple_of(off[k], 128), 128)],
                    slab_v.at[t * 16 + k], sems[k])
            for k in range(16):
                pltpu.make_async_copy(
                    tab_hbm.at[:, pl.ds(0, 128)],
                    slab_v.at[t * 16 + k], sems[k]).wait()
                v = plsc.load_gather(
                    slab_v.at[t * 16 + k],
                    [lanes, jnp.full((16,), col[k], jnp.int32)])
                plsc.store_scatter(flat_v, [(g * 16 + k) * EMBED + lanes], v)
            return carry

        lax.fori_loop(0, NGRP, grp, 0)

    pltpu.sync_copy(uflat_v, uout_hbm.at[pl.ds(base * EMBED, BPW * EMBED)])
    pltpu.sync_copy(iflat_v, iout_hbm.at[pl.ds(base * EMBED, BPW * EMBED)])


@functools.cache
def _gather():
    return pl.kernel(
        _gather_body,
        mesh=plsc.VectorSubcoreMesh(core_axis_name="c", subcore_axis_name="s"),
        compiler_params=pltpu.CompilerParams(needs_layout_passes=False),
        out_type=[
            jax.ShapeDtypeStruct((BATCH * EMBED,), jnp.float32),
            jax.ShapeDtypeStruct((BATCH * EMBED,), jnp.float32),
        ],
        scratch_types=(
            [pltpu.VMEM((BPW,), jnp.int32),
             pltpu.VMEM((BPW,), jnp.int32),
             pltpu.VMEM((4 * DEPTH, EMBED, 128), jnp.float32),
             pltpu.VMEM((BPW * EMBED,), jnp.float32),
             pltpu.VMEM((BPW * EMBED,), jnp.float32)]
            + [pltpu.SemaphoreType.DMA] * (2 * DEPTH)
        ),
    )


B_BLK = 16384
R_BLK = B_BLK // 8


def _mlp_body(xu_ref, xi_ref, m1u_ref, m1i_ref, b1_ref, m2_ref, b2_ref,
              m3_ref, b3_ref, out_ref):
    h = (jnp.dot(xu_ref[...], m1u_ref[...], preferred_element_type=jnp.float32)
         + jnp.dot(xi_ref[...], m1i_ref[...], preferred_element_type=jnp.float32)
         + b1_ref[...])
    h = jnp.maximum(h, 0.0)
    h = jnp.dot(h, m2_ref[...], preferred_element_type=jnp.float32) + b2_ref[...]
    h = jnp.maximum(h, 0.0)
    out_ref[...] = (jnp.dot(h, m3_ref[...], preferred_element_type=jnp.float32)
                    + b3_ref[...])


def _mlp(xu, xi, m1u, m1i, b1t, m2, b2t, m3, b3t):
    grid = (BATCH // B_BLK,)
    full = lambda shape: pl.BlockSpec(shape, lambda i: (0, 0))
    return pl.pallas_call(
        _mlp_body,
        grid=grid,
        in_specs=[
            pl.BlockSpec((R_BLK, 128), lambda i: (i, 0)),
            pl.BlockSpec((R_BLK, 128), lambda i: (i, 0)),
            full((128, 128)),
            full((128, 128)),
            full((1, 128)),
            full((128, 64)),
            full((1, 64)),
            full((64, 8)),
            full((1, 8)),
        ],
        out_specs=pl.BlockSpec((R_BLK, 8), lambda i: (i, 0)),
        out_shape=jax.ShapeDtypeStruct((BATCH // 8, 8), jnp.float32),
    )(xu, xi, m1u, m1i, b1t, m2, b2t, m3, b3t)


def kernel(user, item, user_table, item_table, W1, b1, W2, b2, W3, b3):
    u_flat, i_flat = _gather()(user.astype(jnp.int32), item.astype(jnp.int32),
                               user_table.T, item_table.T)
    xu = u_flat.reshape(BATCH // 8, 128)
    xi = i_flat.reshape(BATCH // 8, 128)
    eye8 = jnp.eye(8, dtype=jnp.float32)
    m1u = jnp.kron(eye8, W1[:EMBED])
    m1i = jnp.kron(eye8, W1[EMBED:])
    b1t = jnp.tile(b1, 8).reshape(1, 128)
    m2 = jnp.kron(eye8, W2)
    b2t = jnp.tile(b2, 8).reshape(1, 64)
    m3 = jnp.kron(eye8, W3)
    b3t = jnp.tile(b3, 8).reshape(1, 8)
    out = _mlp(xu, xi, m1u, m1i, b1t, m2, b2t, m3, b3t)
    return out.reshape(BATCH)
